# Initial kernel scaffold; baseline (speedup 1.0000x reference)
import jax
import jax.numpy as jnp
from jax.experimental import pallas as pl


def kernel(pos_rw, neg_rw, embedding):
    raise NotImplementedError("write your pallas kernel here")


# speedup vs baseline: 2.8828x; 2.8828x over previous
import functools

import jax
import jax.numpy as jnp
from jax.experimental import pallas as pl
from jax.experimental.pallas import tpu as pltpu
from jax.experimental.pallas import tpu_sc as plsc

_NUM_NODES = 100000
_D = 128
_B = 102400
_CTX = 10
_EPS = 1e-15

_NUM_IDX = 2 * _B * _CTX
_GATHER_WINDOW = 128
_W = 512


def _sc_gather(embedding, ids):
    mesh = plsc.VectorSubcoreMesh(core_axis_name="c", subcore_axis_name="s")

    @functools.partial(
        pl.kernel,
        out_type=jax.ShapeDtypeStruct((_NUM_IDX, _D), jnp.float32),
        mesh=mesh,
    )
    def gather_kernel(x_hbm, i_hbm, o_hbm):
        def body(i_vmem, o_vmem):
            pltpu.sync_copy(x_hbm.at[i_vmem.at[0]], o_vmem)

        pltpu.emit_pipeline(
            body,
            grid=(_NUM_IDX // _GATHER_WINDOW,),
            in_specs=[
                pl.BlockSpec((1, _GATHER_WINDOW), index_map=lambda i: (0, i))
            ],
            out_specs=[
                pl.BlockSpec((_GATHER_WINDOW, _D), index_map=lambda i: (i, 0))
            ],
            core_axis_name=("c", "s"),
            dimension_semantics=(pltpu.PARALLEL,),
        )(i_hbm, o_hbm)

    return gather_kernel(embedding, ids.reshape(1, _NUM_IDX))


def _loss_body(pos_ref, neg_ref, out_ref):
    dots_p = jnp.sum(pos_ref[:, :1, :] * pos_ref[:, 1:, :], axis=-1)
    dots_n = jnp.sum(neg_ref[:, :1, :] * neg_ref[:, 1:, :], axis=-1)
    term_p = -jnp.log(jnp.maximum(jax.nn.sigmoid(dots_p), 0.0) + _EPS)
    term_n = -jnp.log(jnp.maximum(1.0 - jax.nn.sigmoid(dots_n), 0.0) + _EPS)
    part = (jnp.sum(term_p) + jnp.sum(term_n)).reshape(1, 1)

    @pl.when(pl.program_id(0) == 0)
    def _():
        out_ref[...] = jnp.zeros((1, 1), jnp.float32)

    out_ref[...] += part


def _tc_loss(gathered):
    g3 = gathered.reshape(2 * _B, _CTX, _D)
    nblk = _B // _W
    out = pl.pallas_call(
        _loss_body,
        grid=(nblk,),
        in_specs=[
            pl.BlockSpec((_W, _CTX, _D), lambda i: (i, 0, 0)),
            pl.BlockSpec((_W, _CTX, _D), lambda i, n=nblk: (i + n, 0, 0)),
        ],
        out_specs=pl.BlockSpec((1, 1), lambda i: (0, 0)),
        out_shape=jax.ShapeDtypeStruct((1, 1), jnp.float32),
    )(g3, g3)
    return out[0, 0]


def kernel(pos_rw, neg_rw, embedding):
    ids = jnp.concatenate(
        [pos_rw.reshape(-1), neg_rw.reshape(-1)]
    ).astype(jnp.int32)
    gathered = _sc_gather(embedding, ids)
    total = _tc_loss(gathered)
    return total / jnp.float32(_B * (_CTX - 1))


# speedup vs baseline: 5.5108x; 1.9116x over previous
import functools

import jax
import jax.numpy as jnp
from jax import lax
from jax.experimental import pallas as pl
from jax.experimental.pallas import tpu as pltpu
from jax.experimental.pallas import tpu_sc as plsc

_NUM_NODES = 100000
_D = 128
_B = 102400
_CTX = 10
_NPAIR = _CTX - 1
_EPS = 1e-15

_NW = 32
_WALKS = 2 * _B
_WPC = 16
_IDS_PER_CHUNK = _WPC * _CTX
_GRP = 80
_NGRP = _IDS_PER_CHUNK // _GRP
_CHUNKS = _WALKS // (_NW * _WPC)
_OUT_ROWS = _WPC * _NPAIR


def _sc_partial_dots(embedding, ids3d):
    mesh = plsc.VectorSubcoreMesh(core_axis_name="c", subcore_axis_name="s")

    @functools.partial(
        pl.kernel,
        out_type=jax.ShapeDtypeStruct((_WALKS * _NPAIR, 16), jnp.float32),
        mesh=mesh,
        scratch_types=[
            pltpu.VMEM((_NGRP, _GRP), jnp.int32),
            pltpu.VMEM((_NGRP, _GRP), jnp.int32),
            pltpu.VMEM((_IDS_PER_CHUNK, _D), jnp.float32),
            pltpu.VMEM((_IDS_PER_CHUNK, _D), jnp.float32),
            pltpu.VMEM((_OUT_ROWS, 16), jnp.float32),
            pltpu.VMEM((_OUT_ROWS, 16), jnp.float32),
            pltpu.SemaphoreType.DMA,
            pltpu.SemaphoreType.DMA,
            pltpu.SemaphoreType.DMA,
            pltpu.SemaphoreType.DMA,
            pltpu.SemaphoreType.DMA,
            pltpu.SemaphoreType.DMA,
        ],
    )
    def sc_kernel(table_hbm, ids_hbm, out_hbm,
                  idx0, idx1, rows0, rows1, ob0, ob1,
                  isem0, isem1, rsem0, rsem1, osem0, osem1):
        wid = lax.axis_index("s") * 2 + lax.axis_index("c")
        cc0 = wid * _CHUNKS

        idxb = (idx0, idx1)
        rowb = (rows0, rows1)
        outb = (ob0, ob1)
        isem = (isem0, isem1)
        rsem = (rsem0, rsem1)
        osem = (osem0, osem1)

        def start_gathers(b, _):
            for grp in range(_NGRP):
                pltpu.async_copy(
                    table_hbm.at[idxb[b].at[grp]],
                    rowb[b].at[pl.ds(grp * _GRP, _GRP)],
                    rsem[b],
                )

        def wait_gathers(b):
            for grp in range(_NGRP):
                pltpu.make_async_copy(
                    table_hbm.at[idxb[b].at[grp]],
                    rowb[b].at[pl.ds(grp * _GRP, _GRP)],
                    rsem[b],
                ).wait()

        def out_slice(c):
            return out_hbm.at[pl.ds((cc0 + c) * _OUT_ROWS, _OUT_ROWS)]

        def compute(b):
            rows = rowb[b]
            out = outb[b]

            @pl.loop(0, _WPC)
            def _(w):
                base = w * _CTX
                s = [rows[base, pl.ds(k * 16, 16)] for k in range(8)]
                for j in range(_NPAIR):
                    r = base + 1 + j
                    acc = s[0] * rows[r, pl.ds(0, 16)]
                    for k in range(1, 8):
                        acc = acc + s[k] * rows[r, pl.ds(k * 16, 16)]
                    out[w * _NPAIR + j, :] = acc

        def process(c, b):
            wait_gathers(b)

            @pl.when(c + 2 < _CHUNKS)
            def _(c=c, b=b):
                pltpu.async_copy(ids_hbm.at[cc0 + c + 2], idxb[b], isem[b])

            @pl.when(c >= 2)
            def _(c=c, b=b):
                pltpu.make_async_copy(outb[b], out_slice(c), osem[b]).wait()

            compute(b)
            pltpu.async_copy(outb[b], out_slice(c), osem[b])

            @pl.when(c + 2 < _CHUNKS)
            def _(c=c, b=b):
                pltpu.make_async_copy(
                    ids_hbm.at[cc0 + c + 2], idxb[b], isem[b]
                ).wait()
                start_gathers(b, None)

        pltpu.sync_copy(ids_hbm.at[cc0], idx0)
        pltpu.sync_copy(ids_hbm.at[cc0 + 1], idx1)
        start_gathers(0, None)
        start_gathers(1, None)

        @pl.loop(0, _CHUNKS, step=2)
        def _(c):
            process(c, 0)
            process(c + 1, 1)

        pltpu.make_async_copy(ob0, out_slice(_CHUNKS - 2), osem0).wait()
        pltpu.make_async_copy(ob1, out_slice(_CHUNKS - 1), osem1).wait()

    return sc_kernel(embedding, ids3d)


def _tc_loss_body(pos_ref, neg_ref, out_ref):
    lane = lax.broadcasted_iota(jnp.int32, (_D, 8), 0)
    seg = lax.broadcasted_iota(jnp.int32, (_D, 8), 1)
    fold = (lane // 16 == seg).astype(jnp.float32)

    dn = (((1,), (0,)), ((), ()))
    dots_p = lax.dot_general(pos_ref[...], fold, dn,
                             preferred_element_type=jnp.float32)
    dots_n = lax.dot_general(neg_ref[...], fold, dn,
                             preferred_element_type=jnp.float32)
    term_p = -jnp.log(jnp.maximum(jax.nn.sigmoid(dots_p), 0.0) + _EPS)
    term_n = -jnp.log(jnp.maximum(1.0 - jax.nn.sigmoid(dots_n), 0.0) + _EPS)
    part = (jnp.sum(term_p) + jnp.sum(term_n)).reshape(1, 1)

    @pl.when(pl.program_id(0) == 0)
    def _():
        out_ref[...] = jnp.zeros((1, 1), jnp.float32)

    out_ref[...] += part


def _tc_loss(partials):
    rows_total = _WALKS * _NPAIR // 8
    half = rows_total // 2
    p2 = partials.reshape(rows_total, _D)
    rb = 1152
    nblk = half // rb
    out = pl.pallas_call(
        _tc_loss_body,
        grid=(nblk,),
        in_specs=[
            pl.BlockSpec((rb, _D), lambda i: (i, 0)),
            pl.BlockSpec((rb, _D), lambda i, n=nblk: (i + n, 0)),
        ],
        out_specs=pl.BlockSpec((1, 1), lambda i: (0, 0)),
        out_shape=jax.ShapeDtypeStruct((1, 1), jnp.float32),
    )(p2, p2)
    return out[0, 0]


def kernel(pos_rw, neg_rw, embedding):
    ids = jnp.concatenate(
        [pos_rw.reshape(-1), neg_rw.reshape(-1)]
    ).astype(jnp.int32)
    ids3d = ids.reshape(_NW * _CHUNKS, _NGRP, _GRP)
    partials = _sc_partial_dots(embedding, ids3d)
    total = _tc_loss(partials)
    return total / jnp.float32(_B * _NPAIR)


# speedup vs baseline: 6.1375x; 1.1137x over previous
import dataclasses
import functools

import jax
import jax.numpy as jnp
from jax import lax
from jax.experimental import pallas as pl
from jax.experimental.pallas import tpu as pltpu
from jax.experimental.pallas import tpu_sc as plsc

_NUM_NODES = 100000
_D = 128
_B = 102400
_CTX = 10
_NPAIR = _CTX - 1
_EPS = 1e-15

_NW = 32
_WALKS = 2 * _B
_WPC = 16
_IDS_PER_CHUNK = _WPC * _CTX
_GRP = 80
_NGRP = _IDS_PER_CHUNK // _GRP
_CHUNKS = _WALKS // (_NW * _WPC)
_OUT_ROWS = _WPC * _NPAIR


def _xlane_gather(v, idx):
    dnums = lax.GatherDimensionNumbers(
        offset_dims=(), collapsed_slice_dims=(0,), start_index_map=(0,)
    )
    return lax.gather(
        v, idx[:, None], dnums, (1,),
        mode=lax.GatherScatterMode.PROMISE_IN_BOUNDS,
    )


def _sc_dots(embedding, ids3d):
    mesh = plsc.VectorSubcoreMesh(core_axis_name="c", subcore_axis_name="s")
    cp = pltpu.CompilerParams()
    if "needs_layout_passes" in pltpu.CompilerParams.__dataclass_fields__:
        cp = dataclasses.replace(cp, needs_layout_passes=False)

    @functools.partial(
        pl.kernel,
        out_type=jax.ShapeDtypeStruct((_WALKS * _NPAIR,), jnp.float32),
        mesh=mesh,
        compiler_params=cp,
        scratch_types=[
            pltpu.VMEM((_NGRP, _GRP), jnp.int32),
            pltpu.VMEM((_NGRP, _GRP), jnp.int32),
            pltpu.VMEM((_IDS_PER_CHUNK, _D), jnp.float32),
            pltpu.VMEM((_IDS_PER_CHUNK, _D), jnp.float32),
            pltpu.VMEM((_OUT_ROWS, 16), jnp.float32),
            pltpu.VMEM((_OUT_ROWS,), jnp.float32),
            pltpu.VMEM((_OUT_ROWS,), jnp.float32),
            pltpu.SemaphoreType.DMA,
            pltpu.SemaphoreType.DMA,
            pltpu.SemaphoreType.DMA,
            pltpu.SemaphoreType.DMA,
            pltpu.SemaphoreType.DMA,
            pltpu.SemaphoreType.DMA,
        ],
    )
    def sc_kernel(table_hbm, ids_hbm, out_hbm,
                  idx0, idx1, rows0, rows1, staged, ob0, ob1,
                  isem0, isem1, rsem0, rsem1, osem0, osem1):
        wid = lax.axis_index("s") * 2 + lax.axis_index("c")
        cc0 = wid * _CHUNKS

        idxb = (idx0, idx1)
        rowb = (rows0, rows1)
        outb = (ob0, ob1)
        isem = (isem0, isem1)
        rsem = (rsem0, rsem1)
        osem = (osem0, osem1)

        def start_gathers(b, _):
            for grp in range(_NGRP):
                pltpu.async_copy(
                    table_hbm.at[idxb[b].at[grp]],
                    rowb[b].at[pl.ds(grp * _GRP, _GRP)],
                    rsem[b],
                )

        def wait_gathers(b):
            for grp in range(_NGRP):
                pltpu.make_async_copy(
                    table_hbm.at[idxb[b].at[grp]],
                    rowb[b].at[pl.ds(grp * _GRP, _GRP)],
                    rsem[b],
                ).wait()

        def out_slice(c):
            return out_hbm.at[pl.ds((cc0 + c) * _OUT_ROWS, _OUT_ROWS)]

        lane = lax.broadcasted_iota(jnp.int32, (16,), 0)
        bfly = [lane ^ s for s in (1, 2, 4, 8)]

        def compute(b):
            rows = rowb[b]
            out = outb[b]

            @pl.loop(0, _WPC)
            def _(w):
                base = w * _CTX
                s = [rows[base, pl.ds(k * 16, 16)] for k in range(8)]
                for j in range(_NPAIR):
                    r = base + 1 + j
                    acc = s[0] * rows[r, pl.ds(0, 16)]
                    for k in range(1, 8):
                        acc = acc + s[k] * rows[r, pl.ds(k * 16, 16)]
                    for p in bfly:
                        acc = acc + _xlane_gather(acc, p)
                    staged[w * _NPAIR + j, :] = acc

            @pl.loop(0, _OUT_ROWS // 16)
            def _(g):
                vals = plsc.load_gather(staged, [g * 16 + lane, lane])
                out[pl.ds(g * 16, 16)] = vals

        def process(c, b):
            wait_gathers(b)

            @pl.when(c + 2 < _CHUNKS)
            def _(c=c, b=b):
                pltpu.async_copy(ids_hbm.at[cc0 + c + 2], idxb[b], isem[b])

            @pl.when(c >= 2)
            def _(c=c, b=b):
                pltpu.make_async_copy(outb[b], out_slice(c), osem[b]).wait()

            compute(b)
            pltpu.async_copy(outb[b], out_slice(c), osem[b])

            @pl.when(c + 2 < _CHUNKS)
            def _(c=c, b=b):
                pltpu.make_async_copy(
                    ids_hbm.at[cc0 + c + 2], idxb[b], isem[b]
                ).wait()
                start_gathers(b, None)

        pltpu.sync_copy(ids_hbm.at[cc0], idx0)
        pltpu.sync_copy(ids_hbm.at[cc0 + 1], idx1)
        start_gathers(0, None)
        start_gathers(1, None)

        @pl.loop(0, _CHUNKS, step=2)
        def _(c):
            process(c, 0)
            process(c + 1, 1)

        pltpu.make_async_copy(ob0, out_slice(_CHUNKS - 2), osem0).wait()
        pltpu.make_async_copy(ob1, out_slice(_CHUNKS - 1), osem1).wait()

    return sc_kernel(embedding, ids3d)


def _tc_loss_body(pos_ref, neg_ref, out_ref):
    dots_p = pos_ref[...]
    dots_n = neg_ref[...]
    term_p = -jnp.log(jnp.maximum(jax.nn.sigmoid(dots_p), 0.0) + _EPS)
    term_n = -jnp.log(jnp.maximum(1.0 - jax.nn.sigmoid(dots_n), 0.0) + _EPS)
    part = (jnp.sum(term_p) + jnp.sum(term_n)).reshape(1, 1)

    @pl.when(pl.program_id(0) == 0)
    def _():
        out_ref[...] = jnp.zeros((1, 1), jnp.float32)

    out_ref[...] += part


def _tc_loss(dots):
    rows_total = _WALKS * _NPAIR // _D
    half = rows_total // 2
    d2 = dots.reshape(rows_total, _D)
    rb = 720
    nblk = half // rb
    out = pl.pallas_call(
        _tc_loss_body,
        grid=(nblk,),
        in_specs=[
            pl.BlockSpec((rb, _D), lambda i: (i, 0)),
            pl.BlockSpec((rb, _D), lambda i, n=nblk: (i + n, 0)),
        ],
        out_specs=pl.BlockSpec((1, 1), lambda i: (0, 0)),
        out_shape=jax.ShapeDtypeStruct((1, 1), jnp.float32),
    )(d2, d2)
    return out[0, 0]


def kernel(pos_rw, neg_rw, embedding):
    ids = jnp.concatenate(
        [pos_rw.reshape(-1), neg_rw.reshape(-1)]
    ).astype(jnp.int32)
    ids3d = ids.reshape(_NW * _CHUNKS, _NGRP, _GRP)
    dots = _sc_dots(embedding, ids3d)
    total = _tc_loss(dots)
    return total / jnp.float32(_B * _NPAIR)


# speedup vs baseline: 6.7488x; 1.0996x over previous
import dataclasses
import functools

import jax
import jax.numpy as jnp
from jax import lax
from jax.experimental import pallas as pl
from jax.experimental.pallas import tpu as pltpu
from jax.experimental.pallas import tpu_sc as plsc

_NUM_NODES = 100000
_D = 128
_B = 102400
_CTX = 10
_NPAIR = _CTX - 1
_EPS = 1e-15

_NW = 32
_WALKS = 2 * _B
_WPC = 16
_IDS_PER_CHUNK = _WPC * _CTX
_GRP = 80
_NGRP = _IDS_PER_CHUNK // _GRP
_CHUNKS = _WALKS // (_NW * _WPC)
_OUT_ROWS = _WPC * _NPAIR


def _xlane_gather(v, idx):
    dnums = lax.GatherDimensionNumbers(
        offset_dims=(), collapsed_slice_dims=(0,), start_index_map=(0,)
    )
    return lax.gather(
        v, idx[:, None], dnums, (1,),
        mode=lax.GatherScatterMode.PROMISE_IN_BOUNDS,
    )


def _sc_dots(embedding, ids3d):
    mesh = plsc.VectorSubcoreMesh(core_axis_name="c", subcore_axis_name="s")
    cp = pltpu.CompilerParams()
    if "needs_layout_passes" in pltpu.CompilerParams.__dataclass_fields__:
        cp = dataclasses.replace(cp, needs_layout_passes=False)

    @functools.partial(
        pl.kernel,
        out_type=jax.ShapeDtypeStruct((_WALKS * _NPAIR,), jnp.float32),
        mesh=mesh,
        compiler_params=cp,
        scratch_types=[
            pltpu.VMEM((_NGRP, _GRP), jnp.int32),
            pltpu.VMEM((_NGRP, _GRP), jnp.int32),
            pltpu.VMEM((_IDS_PER_CHUNK, _D), jnp.float32),
            pltpu.VMEM((_IDS_PER_CHUNK, _D), jnp.float32),
            pltpu.VMEM((_OUT_ROWS, 16), jnp.float32),
            pltpu.VMEM((_OUT_ROWS,), jnp.float32),
            pltpu.VMEM((_OUT_ROWS,), jnp.float32),
            pltpu.SemaphoreType.DMA,
            pltpu.SemaphoreType.DMA,
            pltpu.SemaphoreType.DMA,
            pltpu.SemaphoreType.DMA,
            pltpu.SemaphoreType.DMA,
            pltpu.SemaphoreType.DMA,
        ],
    )
    def sc_kernel(table_hbm, ids_hbm, out_hbm,
                  idx0, idx1, rows0, rows1, staged, ob0, ob1,
                  isem0, isem1, rsem0, rsem1, osem0, osem1):
        wid = lax.axis_index("s") * 2 + lax.axis_index("c")
        cc0 = wid * _CHUNKS

        idxb = (idx0, idx1)
        rowb = (rows0, rows1)
        outb = (ob0, ob1)
        isem = (isem0, isem1)
        rsem = (rsem0, rsem1)
        osem = (osem0, osem1)

        def start_gathers(b, _):
            for grp in range(_NGRP):
                pltpu.async_copy(
                    table_hbm.at[idxb[b].at[grp]],
                    rowb[b].at[pl.ds(grp * _GRP, _GRP)],
                    rsem[b],
                )

        def wait_gathers(b):
            for grp in range(_NGRP):
                pltpu.make_async_copy(
                    table_hbm.at[idxb[b].at[grp]],
                    rowb[b].at[pl.ds(grp * _GRP, _GRP)],
                    rsem[b],
                ).wait()

        def out_slice(c):
            return out_hbm.at[pl.ds((cc0 + c) * _OUT_ROWS, _OUT_ROWS)]

        lane = lax.broadcasted_iota(jnp.int32, (16,), 0)
        last = lane * 0 + 15

        def compute(b):
            rows = rowb[b]
            out = outb[b]

            @pl.loop(0, _WPC)
            def _(w):
                base = w * _CTX
                s = [rows[base, pl.ds(k * 16, 16)] for k in range(8)]
                for j in range(_NPAIR):
                    r = base + 1 + j
                    acc = s[0] * rows[r, pl.ds(0, 16)]
                    for k in range(1, 8):
                        acc = acc + s[k] * rows[r, pl.ds(k * 16, 16)]
                    staged[w * _NPAIR + j, :] = jnp.cumsum(acc)

            @pl.loop(0, _OUT_ROWS // 16)
            def _(g):
                vals = plsc.load_gather(staged, [g * 16 + lane, last])
                out[pl.ds(g * 16, 16)] = vals

        def process(c, b):
            wait_gathers(b)

            @pl.when(c + 2 < _CHUNKS)
            def _(c=c, b=b):
                pltpu.async_copy(ids_hbm.at[cc0 + c + 2], idxb[b], isem[b])

            @pl.when(c >= 2)
            def _(c=c, b=b):
                pltpu.make_async_copy(outb[b], out_slice(c), osem[b]).wait()

            compute(b)
            pltpu.async_copy(outb[b], out_slice(c), osem[b])

            @pl.when(c + 2 < _CHUNKS)
            def _(c=c, b=b):
                pltpu.make_async_copy(
                    ids_hbm.at[cc0 + c + 2], idxb[b], isem[b]
                ).wait()
                start_gathers(b, None)

        pltpu.sync_copy(ids_hbm.at[cc0], idx0)
        pltpu.sync_copy(ids_hbm.at[cc0 + 1], idx1)
        start_gathers(0, None)
        start_gathers(1, None)

        @pl.loop(0, _CHUNKS, step=2)
        def _(c):
            process(c, 0)
            process(c + 1, 1)

        pltpu.make_async_copy(ob0, out_slice(_CHUNKS - 2), osem0).wait()
        pltpu.make_async_copy(ob1, out_slice(_CHUNKS - 1), osem1).wait()

    return sc_kernel(embedding, ids3d)


def _tc_loss_body(pos_ref, neg_ref, out_ref):
    dots_p = pos_ref[...]
    dots_n = neg_ref[...]
    term_p = -jnp.log(jnp.maximum(jax.nn.sigmoid(dots_p), 0.0) + _EPS)
    term_n = -jnp.log(jnp.maximum(1.0 - jax.nn.sigmoid(dots_n), 0.0) + _EPS)
    part = (jnp.sum(term_p) + jnp.sum(term_n)).reshape(1, 1)

    @pl.when(pl.program_id(0) == 0)
    def _():
        out_ref[...] = jnp.zeros((1, 1), jnp.float32)

    out_ref[...] += part


def _tc_loss(dots):
    rows_total = _WALKS * _NPAIR // _D
    half = rows_total // 2
    d2 = dots.reshape(rows_total, _D)
    rb = 720
    nblk = half // rb
    out = pl.pallas_call(
        _tc_loss_body,
        grid=(nblk,),
        in_specs=[
            pl.BlockSpec((rb, _D), lambda i: (i, 0)),
            pl.BlockSpec((rb, _D), lambda i, n=nblk: (i + n, 0)),
        ],
        out_specs=pl.BlockSpec((1, 1), lambda i: (0, 0)),
        out_shape=jax.ShapeDtypeStruct((1, 1), jnp.float32),
    )(d2, d2)
    return out[0, 0]


def kernel(pos_rw, neg_rw, embedding):
    ids = jnp.concatenate(
        [pos_rw.reshape(-1), neg_rw.reshape(-1)]
    ).astype(jnp.int32)
    ids3d = ids.reshape(_NW * _CHUNKS, _NGRP, _GRP)
    dots = _sc_dots(embedding, ids3d)
    total = _tc_loss(dots)
    return total / jnp.float32(_B * _NPAIR)


# speedup vs baseline: 6.7867x; 1.0056x over previous
import dataclasses
import functools

import jax
import jax.numpy as jnp
from jax import lax
from jax.experimental import pallas as pl
from jax.experimental.pallas import tpu as pltpu
from jax.experimental.pallas import tpu_sc as plsc

_NUM_NODES = 100000
_D = 128
_B = 102400
_CTX = 10
_NPAIR = _CTX - 1
_EPS = 1e-15

_NW = 32
_WALKS = 2 * _B
_WPC = 16
_IDS_PER_CHUNK = _WPC * _CTX
_GRP = 80
_NGRP = _IDS_PER_CHUNK // _GRP
_CHUNKS = _WALKS // (_NW * _WPC)
_OUT_ROWS = _WPC * _NPAIR


def _xlane_gather(v, idx):
    dnums = lax.GatherDimensionNumbers(
        offset_dims=(), collapsed_slice_dims=(0,), start_index_map=(0,)
    )
    return lax.gather(
        v, idx[:, None], dnums, (1,),
        mode=lax.GatherScatterMode.PROMISE_IN_BOUNDS,
    )


def _sc_dots(embedding, ids3d):
    mesh = plsc.VectorSubcoreMesh(core_axis_name="c", subcore_axis_name="s")
    cp = pltpu.CompilerParams()
    if "needs_layout_passes" in pltpu.CompilerParams.__dataclass_fields__:
        cp = dataclasses.replace(cp, needs_layout_passes=False)

    @functools.partial(
        pl.kernel,
        out_type=jax.ShapeDtypeStruct((_WALKS * _NPAIR,), jnp.float32),
        mesh=mesh,
        compiler_params=cp,
        scratch_types=[
            pltpu.VMEM((_NGRP, _GRP), jnp.int32),
            pltpu.VMEM((_NGRP, _GRP), jnp.int32),
            pltpu.VMEM((_IDS_PER_CHUNK, _D), jnp.float32),
            pltpu.VMEM((_IDS_PER_CHUNK, _D), jnp.float32),
            pltpu.VMEM((_OUT_ROWS, 16), jnp.float32),
            pltpu.VMEM((_OUT_ROWS,), jnp.float32),
            pltpu.VMEM((_OUT_ROWS,), jnp.float32),
            pltpu.SemaphoreType.DMA,
            pltpu.SemaphoreType.DMA,
            pltpu.SemaphoreType.DMA,
            pltpu.SemaphoreType.DMA,
            pltpu.SemaphoreType.DMA,
            pltpu.SemaphoreType.DMA,
        ],
    )
    def sc_kernel(table_hbm, ids_hbm, out_hbm,
                  idx0, idx1, rows0, rows1, staged, ob0, ob1,
                  isem0, isem1, rsem0, rsem1, osem0, osem1):
        wid = lax.axis_index("s") * 2 + lax.axis_index("c")
        cc0 = wid * _CHUNKS

        idxb = (idx0, idx1)
        rowb = (rows0, rows1)
        outb = (ob0, ob1)
        isem = (isem0, isem1)
        rsem = (rsem0, rsem1)
        osem = (osem0, osem1)

        def start_gathers(b, _):
            for grp in range(_NGRP):
                pltpu.async_copy(
                    table_hbm.at[idxb[b].at[grp]],
                    rowb[b].at[pl.ds(grp * _GRP, _GRP)],
                    rsem[b],
                )

        def wait_gathers(b):
            for grp in range(_NGRP):
                pltpu.make_async_copy(
                    table_hbm.at[idxb[b].at[grp]],
                    rowb[b].at[pl.ds(grp * _GRP, _GRP)],
                    rsem[b],
                ).wait()

        def out_slice(c):
            return out_hbm.at[pl.ds((cc0 + c) * _OUT_ROWS, _OUT_ROWS)]

        lane = lax.broadcasted_iota(jnp.int32, (16,), 0)

        def compute(b):
            rows = rowb[b]
            out = outb[b]

            @pl.loop(0, _WPC)
            def _(w):
                base = w * _CTX
                s = [rows[base, pl.ds(k * 16, 16)] for k in range(8)]
                for j in range(_NPAIR):
                    r = base + 1 + j
                    acc = s[0] * rows[r, pl.ds(0, 16)]
                    for k in range(1, 8):
                        acc = acc + s[k] * rows[r, pl.ds(k * 16, 16)]
                    staged[w * _NPAIR + j, :] = acc

            @pl.loop(0, _OUT_ROWS // 16)
            def _(g):
                row_idx = g * 16 + lane
                t = [
                    plsc.load_gather(staged, [row_idx, lane * 0 + k])
                    for k in range(16)
                ]
                while len(t) > 1:
                    t = [t[i] + t[i + 1] for i in range(0, len(t), 2)]
                out[pl.ds(g * 16, 16)] = t[0]

        def process(c, b):
            wait_gathers(b)

            @pl.when(c + 2 < _CHUNKS)
            def _(c=c, b=b):
                pltpu.async_copy(ids_hbm.at[cc0 + c + 2], idxb[b], isem[b])

            @pl.when(c >= 2)
            def _(c=c, b=b):
                pltpu.make_async_copy(outb[b], out_slice(c), osem[b]).wait()

            compute(b)
            pltpu.async_copy(outb[b], out_slice(c), osem[b])

            @pl.when(c + 2 < _CHUNKS)
            def _(c=c, b=b):
                pltpu.make_async_copy(
                    ids_hbm.at[cc0 + c + 2], idxb[b], isem[b]
                ).wait()
                start_gathers(b, None)

        pltpu.sync_copy(ids_hbm.at[cc0], idx0)
        pltpu.sync_copy(ids_hbm.at[cc0 + 1], idx1)
        start_gathers(0, None)
        start_gathers(1, None)

        @pl.loop(0, _CHUNKS, step=2)
        def _(c):
            process(c, 0)
            process(c + 1, 1)

        pltpu.make_async_copy(ob0, out_slice(_CHUNKS - 2), osem0).wait()
        pltpu.make_async_copy(ob1, out_slice(_CHUNKS - 1), osem1).wait()

    return sc_kernel(embedding, ids3d)


def _tc_loss_body(pos_ref, neg_ref, out_ref):
    dots_p = pos_ref[...]
    dots_n = neg_ref[...]
    term_p = -jnp.log(jnp.maximum(jax.nn.sigmoid(dots_p), 0.0) + _EPS)
    term_n = -jnp.log(jnp.maximum(1.0 - jax.nn.sigmoid(dots_n), 0.0) + _EPS)
    part = (jnp.sum(term_p) + jnp.sum(term_n)).reshape(1, 1)

    @pl.when(pl.program_id(0) == 0)
    def _():
        out_ref[...] = jnp.zeros((1, 1), jnp.float32)

    out_ref[...] += part


def _tc_loss(dots):
    rows_total = _WALKS * _NPAIR // _D
    half = rows_total // 2
    d2 = dots.reshape(rows_total, _D)
    rb = 720
    nblk = half // rb
    out = pl.pallas_call(
        _tc_loss_body,
        grid=(nblk,),
        in_specs=[
            pl.BlockSpec((rb, _D), lambda i: (i, 0)),
            pl.BlockSpec((rb, _D), lambda i, n=nblk: (i + n, 0)),
        ],
        out_specs=pl.BlockSpec((1, 1), lambda i: (0, 0)),
        out_shape=jax.ShapeDtypeStruct((1, 1), jnp.float32),
    )(d2, d2)
    return out[0, 0]


def kernel(pos_rw, neg_rw, embedding):
    ids = jnp.concatenate(
        [pos_rw.reshape(-1), neg_rw.reshape(-1)]
    ).astype(jnp.int32)
    ids3d = ids.reshape(_NW * _CHUNKS, _NGRP, _GRP)
    dots = _sc_dots(embedding, ids3d)
    total = _tc_loss(dots)
    return total / jnp.float32(_B * _NPAIR)


# speedup vs baseline: 13.9606x; 2.0571x over previous
import dataclasses
import functools

import jax
import jax.numpy as jnp
from jax import lax
from jax.experimental import pallas as pl
from jax.experimental.pallas import tpu as pltpu
from jax.experimental.pallas import tpu_sc as plsc

_NUM_NODES = 100000
_D = 128
_B = 102400
_CTX = 10
_NPAIR = _CTX - 1
_EPS = 1e-15

_NW = 32
_WALKS = 2 * _B
_WPC = 16
_IDS_PER_CHUNK = _WPC * _CTX
_GRP = 80
_NGRP = _IDS_PER_CHUNK // _GRP
_CHUNKS = _WALKS // (_NW * _WPC)
_OUT_ROWS = _WPC * _NPAIR


def _xlane_gather(v, idx):
    dnums = lax.GatherDimensionNumbers(
        offset_dims=(), collapsed_slice_dims=(0,), start_index_map=(0,)
    )
    return lax.gather(
        v, idx[:, None], dnums, (1,),
        mode=lax.GatherScatterMode.PROMISE_IN_BOUNDS,
    )


def _sc_dots(embedding, ids3d):
    mesh = plsc.VectorSubcoreMesh(core_axis_name="c", subcore_axis_name="s")
    cp = pltpu.CompilerParams()
    if "needs_layout_passes" in pltpu.CompilerParams.__dataclass_fields__:
        cp = dataclasses.replace(cp, needs_layout_passes=False)

    @functools.partial(
        pl.kernel,
        out_type=jax.ShapeDtypeStruct((_WALKS * _NPAIR,), jnp.float32),
        mesh=mesh,
        compiler_params=cp,
        scratch_types=[
            pltpu.VMEM((_NGRP, _GRP), jnp.int32),
            pltpu.VMEM((_NGRP, _GRP), jnp.int32),
            pltpu.VMEM((_IDS_PER_CHUNK, _D), jnp.float32),
            pltpu.VMEM((_IDS_PER_CHUNK, _D), jnp.float32),
            pltpu.VMEM((_OUT_ROWS, 16), jnp.float32),
            pltpu.VMEM((_OUT_ROWS,), jnp.float32),
            pltpu.VMEM((_OUT_ROWS,), jnp.float32),
            pltpu.SemaphoreType.DMA,
            pltpu.SemaphoreType.DMA,
            pltpu.SemaphoreType.DMA,
            pltpu.SemaphoreType.DMA,
            pltpu.SemaphoreType.DMA,
            pltpu.SemaphoreType.DMA,
        ],
    )
    def sc_kernel(table_hbm, ids_hbm, out_hbm,
                  idx0, idx1, rows0, rows1, staged, ob0, ob1,
                  isem0, isem1, rsem0, rsem1, osem0, osem1):
        wid = lax.axis_index("s") * 2 + lax.axis_index("c")
        cc0 = wid * _CHUNKS

        idxb = (idx0, idx1)
        rowb = (rows0, rows1)
        outb = (ob0, ob1)
        isem = (isem0, isem1)
        rsem = (rsem0, rsem1)
        osem = (osem0, osem1)

        def start_gathers(b, _):
            for grp in range(_NGRP):
                pltpu.async_copy(
                    table_hbm.at[idxb[b].at[grp]],
                    rowb[b].at[pl.ds(grp * _GRP, _GRP)],
                    rsem[b],
                )

        def wait_gathers(b):
            for grp in range(_NGRP):
                pltpu.make_async_copy(
                    table_hbm.at[idxb[b].at[grp]],
                    rowb[b].at[pl.ds(grp * _GRP, _GRP)],
                    rsem[b],
                ).wait()

        def out_slice(c):
            return out_hbm.at[pl.ds((cc0 + c) * _OUT_ROWS, _OUT_ROWS)]

        lane = lax.broadcasted_iota(jnp.int32, (16,), 0)

        def compute(b):
            rows = rowb[b]
            out = outb[b]

            @functools.partial(plsc.parallel_loop, 0, _WPC, unroll=2)
            def _(w):
                base = w * _CTX
                s = [rows[base, pl.ds(k * 16, 16)] for k in range(8)]
                for j in range(_NPAIR):
                    r = base + 1 + j
                    t = [s[k] * rows[r, pl.ds(k * 16, 16)] for k in range(8)]
                    while len(t) > 1:
                        t = [t[i] + t[i + 1] for i in range(0, len(t), 2)]
                    staged[w * _NPAIR + j, :] = t[0]

            @functools.partial(plsc.parallel_loop, 0, _OUT_ROWS // 16)
            def _(g):
                row_idx = g * 16 + lane
                t = [
                    plsc.load_gather(staged, [row_idx, lane * 0 + k])
                    for k in range(16)
                ]
                while len(t) > 1:
                    t = [t[i] + t[i + 1] for i in range(0, len(t), 2)]
                out[pl.ds(g * 16, 16)] = t[0]

        def process(c, b):
            wait_gathers(b)

            @pl.when(c + 2 < _CHUNKS)
            def _(c=c, b=b):
                pltpu.async_copy(ids_hbm.at[cc0 + c + 2], idxb[b], isem[b])

            @pl.when(c >= 2)
            def _(c=c, b=b):
                pltpu.make_async_copy(outb[b], out_slice(c), osem[b]).wait()

            compute(b)
            pltpu.async_copy(outb[b], out_slice(c), osem[b])

            @pl.when(c + 2 < _CHUNKS)
            def _(c=c, b=b):
                pltpu.make_async_copy(
                    ids_hbm.at[cc0 + c + 2], idxb[b], isem[b]
                ).wait()
                start_gathers(b, None)

        pltpu.sync_copy(ids_hbm.at[cc0], idx0)
        pltpu.sync_copy(ids_hbm.at[cc0 + 1], idx1)
        start_gathers(0, None)
        start_gathers(1, None)

        @pl.loop(0, _CHUNKS, step=2)
        def _(c):
            process(c, 0)
            process(c + 1, 1)

        pltpu.make_async_copy(ob0, out_slice(_CHUNKS - 2), osem0).wait()
        pltpu.make_async_copy(ob1, out_slice(_CHUNKS - 1), osem1).wait()

    return sc_kernel(embedding, ids3d)


def _tc_loss_body(pos_ref, neg_ref, out_ref):
    dots_p = pos_ref[...]
    dots_n = neg_ref[...]
    term_p = -jnp.log(jnp.maximum(jax.nn.sigmoid(dots_p), 0.0) + _EPS)
    term_n = -jnp.log(jnp.maximum(1.0 - jax.nn.sigmoid(dots_n), 0.0) + _EPS)
    part = (jnp.sum(term_p) + jnp.sum(term_n)).reshape(1, 1)

    @pl.when(pl.program_id(0) == 0)
    def _():
        out_ref[...] = jnp.zeros((1, 1), jnp.float32)

    out_ref[...] += part


def _tc_loss(dots):
    rows_total = _WALKS * _NPAIR // _D
    half = rows_total // 2
    d2 = dots.reshape(rows_total, _D)
    rb = 720
    nblk = half // rb
    out = pl.pallas_call(
        _tc_loss_body,
        grid=(nblk,),
        in_specs=[
            pl.BlockSpec((rb, _D), lambda i: (i, 0)),
            pl.BlockSpec((rb, _D), lambda i, n=nblk: (i + n, 0)),
        ],
        out_specs=pl.BlockSpec((1, 1), lambda i: (0, 0)),
        out_shape=jax.ShapeDtypeStruct((1, 1), jnp.float32),
    )(d2, d2)
    return out[0, 0]


def kernel(pos_rw, neg_rw, embedding):
    ids = jnp.concatenate(
        [pos_rw.reshape(-1), neg_rw.reshape(-1)]
    ).astype(jnp.int32)
    ids3d = ids.reshape(_NW * _CHUNKS, _NGRP, _GRP)
    dots = _sc_dots(embedding, ids3d)
    total = _tc_loss(dots)
    return total / jnp.float32(_B * _NPAIR)
